# baseline (speedup 1.0000x reference)
import functools

import jax
import jax.numpy as jnp
from jax import lax
from jax.experimental import pallas as pl
from jax.experimental.pallas import tpu as pltpu
from jax.experimental.pallas import tpu_sc as plsc

NUM_DEGREE = 1000
D_MODEL = 64
B_TOTAL = 4096 * 200
NC, NS = 2, 16
NW = NC * NS
B_PER_W = B_TOTAL // NW
SUB = 128
CHUNK = 640
N_SUB = CHUNK // SUB
N_CHUNKS = B_PER_W // CHUNK
IDX_ROWS_PER_W = B_PER_W // SUB
NB_ROWS = 2
NB_IDX = 4
N_GROUPS = N_CHUNKS // NB_IDX


@functools.partial(
    pl.kernel,
    out_type=jax.ShapeDtypeStruct((B_TOTAL, D_MODEL), jnp.float32),
    mesh=plsc.VectorSubcoreMesh(core_axis_name="c", subcore_axis_name="s"),
    compiler_params=pltpu.CompilerParams(use_tc_tiling_on_sc=False),
    scratch_types=[
        pltpu.VMEM((NB_IDX, N_SUB, SUB), jnp.int32),
        pltpu.VMEM((NB_ROWS, CHUNK, D_MODEL), jnp.float32),
        [pltpu.SemaphoreType.DMA] * NB_IDX,
        [pltpu.SemaphoreType.DMA] * NB_ROWS,
        [pltpu.SemaphoreType.DMA] * NB_ROWS,
    ],
)
def _gather_kernel(idx_hbm, table_hbm, out_hbm, idx_v, rows_v,
                   sem_idx, sem_gat, sem_out):
    wid = lax.axis_index("s") * NC + lax.axis_index("c")
    idx_row_base = wid * IDX_ROWS_PER_W
    out_base = wid * B_PER_W

    def fire_idx(bi, ci):
        pltpu.async_copy(
            idx_hbm.at[pl.ds(idx_row_base + ci * N_SUB, N_SUB)],
            idx_v.at[bi], sem_idx[bi])

    def wait_idx(bi):
        pltpu.make_async_copy(
            idx_hbm.at[pl.ds(idx_row_base, N_SUB)],
            idx_v.at[bi], sem_idx[bi]).wait()

    def fire_gathers(b, bi):
        for j in range(N_SUB):
            pltpu.async_copy(
                table_hbm.at[idx_v.at[bi].at[j]],
                rows_v.at[b].at[pl.ds(j * SUB, SUB)], sem_gat[b])

    def wait_gathers(b, bi):
        for j in range(N_SUB):
            pltpu.make_async_copy(
                table_hbm.at[idx_v.at[bi].at[j]],
                rows_v.at[b].at[pl.ds(j * SUB, SUB)], sem_gat[b]).wait()

    def fire_out(b, ci):
        pltpu.async_copy(
            rows_v.at[b], out_hbm.at[pl.ds(out_base + ci * CHUNK, CHUNK)],
            sem_out[b])

    def wait_out(b):
        pltpu.make_async_copy(
            rows_v.at[b], out_hbm.at[pl.ds(out_base, CHUNK)],
            sem_out[b]).wait()

    for bi in range(NB_IDX):
        fire_idx(bi, bi)

    def group_body(g, carry):
        for k in range(NB_IDX):
            t = g * NB_IDX + k
            b, bi = k % NB_ROWS, k
            pb, pbi = (k - 1) % NB_ROWS, (k - 1) % NB_IDX
            wait_idx(bi)
            if k >= NB_ROWS:
                wait_out(b)
            else:
                pl.when(g > 0)(lambda b=b: wait_out(b))
            fire_gathers(b, bi)

            def drain_prev(pb=pb, pbi=pbi, t=t):
                wait_gathers(pb, pbi)
                fire_out(pb, t - 1)
                pl.when(t + NB_IDX - 1 < N_CHUNKS)(
                    lambda: fire_idx(pbi, t + NB_IDX - 1))

            if k >= 1:
                drain_prev()
            else:
                pl.when(g > 0)(drain_prev)
        return carry

    lax.fori_loop(0, N_GROUPS, group_body, 0)

    last = N_CHUNKS - 1
    lb, lbi = last % NB_ROWS, last % NB_IDX
    wait_gathers(lb, lbi)
    fire_out(lb, last)
    for b in range(NB_ROWS):
        wait_out(b)


def kernel(degree, degree_encoder_weight):
    idx2d = degree.reshape(B_TOTAL // SUB, SUB)
    out = _gather_kernel(idx2d, degree_encoder_weight)
    return out.reshape(degree.shape[0], degree.shape[1], D_MODEL)


# speedup vs baseline: 1.4048x; 1.4048x over previous
import functools

import jax
import jax.numpy as jnp
from jax import lax
from jax.experimental import pallas as pl
from jax.experimental.pallas import tpu as pltpu
from jax.experimental.pallas import tpu_sc as plsc

NUM_DEGREE = 1000
D_MODEL = 64
B_TOTAL = 4096 * 200
NC, NS = 2, 16
NW = NC * NS
B_PER_W = B_TOTAL // NW
SUB = 128
CHUNK = 640
N_SUB = CHUNK // SUB
N_CHUNKS = B_PER_W // CHUNK
IDX_ROWS_PER_W = B_PER_W // SUB
NB_ROWS = 2
NB_IDX = 4
N_GROUPS = N_CHUNKS // NB_IDX


@functools.partial(
    pl.kernel,
    out_type=jax.ShapeDtypeStruct((B_TOTAL, D_MODEL), jnp.float32),
    mesh=plsc.VectorSubcoreMesh(core_axis_name="c", subcore_axis_name="s"),
    compiler_params=pltpu.CompilerParams(use_tc_tiling_on_sc=False),
    scratch_types=[
        pltpu.VMEM((NB_IDX, N_SUB, SUB), jnp.int32),
        pltpu.VMEM((NB_ROWS, CHUNK, D_MODEL), jnp.float32),
        pltpu.VMEM_SHARED((NUM_DEGREE, D_MODEL), jnp.float32),
        [pltpu.SemaphoreType.DMA] * NB_IDX,
        [pltpu.SemaphoreType.DMA] * NB_ROWS,
        [pltpu.SemaphoreType.DMA] * NB_ROWS,
    ],
)
def _gather_kernel(idx_hbm, table_hbm, out_hbm, idx_v, rows_v, table_sp,
                   sem_idx, sem_gat, sem_out):
    wid = lax.axis_index("s") * NC + lax.axis_index("c")
    idx_row_base = wid * IDX_ROWS_PER_W
    out_base = wid * B_PER_W

    pl.when(lax.axis_index("s") == 0)(
        lambda: pltpu.sync_copy(table_hbm, table_sp))
    plsc.subcore_barrier()

    def fire_idx(bi, ci):
        pltpu.async_copy(
            idx_hbm.at[pl.ds(idx_row_base + ci * N_SUB, N_SUB)],
            idx_v.at[bi], sem_idx[bi])

    def wait_idx(bi):
        pltpu.make_async_copy(
            idx_hbm.at[pl.ds(idx_row_base, N_SUB)],
            idx_v.at[bi], sem_idx[bi]).wait()

    def fire_gathers(b, bi):
        for j in range(N_SUB):
            pltpu.async_copy(
                table_sp.at[idx_v.at[bi].at[j]],
                rows_v.at[b].at[pl.ds(j * SUB, SUB)], sem_gat[b])

    def wait_gathers(b, bi):
        for j in range(N_SUB):
            pltpu.make_async_copy(
                table_sp.at[idx_v.at[bi].at[j]],
                rows_v.at[b].at[pl.ds(j * SUB, SUB)], sem_gat[b]).wait()

    def fire_out(b, ci):
        pltpu.async_copy(
            rows_v.at[b], out_hbm.at[pl.ds(out_base + ci * CHUNK, CHUNK)],
            sem_out[b])

    def wait_out(b):
        pltpu.make_async_copy(
            rows_v.at[b], out_hbm.at[pl.ds(out_base, CHUNK)],
            sem_out[b]).wait()

    for bi in range(NB_IDX):
        fire_idx(bi, bi)

    def group_body(g, carry):
        for k in range(NB_IDX):
            t = g * NB_IDX + k
            b, bi = k % NB_ROWS, k
            pb, pbi = (k - 1) % NB_ROWS, (k - 1) % NB_IDX
            wait_idx(bi)
            if k >= NB_ROWS:
                wait_out(b)
            else:
                pl.when(g > 0)(lambda b=b: wait_out(b))
            fire_gathers(b, bi)

            def drain_prev(pb=pb, pbi=pbi, t=t):
                wait_gathers(pb, pbi)
                fire_out(pb, t - 1)
                pl.when(t + NB_IDX - 1 < N_CHUNKS)(
                    lambda: fire_idx(pbi, t + NB_IDX - 1))

            if k >= 1:
                drain_prev()
            else:
                pl.when(g > 0)(drain_prev)
        return carry

    lax.fori_loop(0, N_GROUPS, group_body, 0)

    last = N_CHUNKS - 1
    lb, lbi = last % NB_ROWS, last % NB_IDX
    wait_gathers(lb, lbi)
    fire_out(lb, last)
    for b in range(NB_ROWS):
        wait_out(b)


def kernel(degree, degree_encoder_weight):
    idx2d = degree.reshape(B_TOTAL // SUB, SUB)
    out = _gather_kernel(idx2d, degree_encoder_weight)
    return out.reshape(degree.shape[0], degree.shape[1], D_MODEL)


# speedup vs baseline: 4.0723x; 2.8988x over previous
import functools

import jax
import jax.numpy as jnp
from jax import lax
from jax.experimental import pallas as pl
from jax.experimental.pallas import tpu as pltpu
from jax.experimental.pallas import tpu_sc as plsc

NUM_DEGREE = 1000
D_MODEL = 64
N_ROWS, N_COLS = 4096, 200
NC, NS = 2, 16
NW = NC * NS
BW = N_ROWS // NW
V_PAD = 1024
TAB_WORDS = D_MODEL * V_PAD
LANES = 16
N_BG = BW // LANES
NB_OUT = 2


@functools.partial(
    pl.kernel,
    out_type=jax.ShapeDtypeStruct((N_COLS, D_MODEL, N_ROWS), jnp.float32),
    mesh=plsc.VectorSubcoreMesh(core_axis_name="c", subcore_axis_name="s"),
    compiler_params=pltpu.CompilerParams(
        use_tc_tiling_on_sc=True, needs_layout_passes=False),
    scratch_types=[
        pltpu.VMEM((N_COLS, BW), jnp.int32),
        pltpu.VMEM((TAB_WORDS,), jnp.float32),
        pltpu.VMEM((NB_OUT, D_MODEL, BW), jnp.float32),
        pltpu.SemaphoreType.DMA,
        pltpu.SemaphoreType.DMA,
        [pltpu.SemaphoreType.DMA] * NB_OUT,
    ],
)
def _gather_kernel(idx_hbm, tab_hbm, out_hbm, idx_all, table_v, out_v,
                   sem_idx, sem_tab, sem_out):
    wid = lax.axis_index("s") * NC + lax.axis_index("c")
    bw = wid * BW

    cp_idx = pltpu.make_async_copy(
        idx_hbm.at[pl.ds(0, N_COLS), pl.ds(bw, BW)], idx_all, sem_idx)
    cp_tab = pltpu.make_async_copy(tab_hbm, table_v, sem_tab)
    cp_idx.start()
    cp_tab.start()
    cp_idx.wait()
    cp_tab.wait()

    def fire_out(b, n):
        pltpu.async_copy(
            out_v.at[b],
            out_hbm.at[n].at[pl.ds(0, D_MODEL), pl.ds(bw, BW)],
            sem_out[b])

    def wait_out(b):
        pltpu.make_async_copy(
            out_v.at[b],
            out_hbm.at[0].at[pl.ds(0, D_MODEL), pl.ds(bw, BW)],
            sem_out[b]).wait()

    def compute_n(n, b):
        pend = []
        for bg in range(N_BG):
            vi = idx_all[n, pl.ds(bg * LANES, LANES)]
            base = ((vi >> 7) << 10) + (vi & 127)
            for c in range(D_MODEL):
                kc = (c // 8) * 8192 + (c % 8) * 128
                g = plsc.load_gather(
                    table_v.at[pl.ds(kc, TAB_WORDS - kc)], [base])
                pend.append((bg, c, g))
                if len(pend) > 2 * LANES:
                    pbg, pc, pg = pend.pop(0)
                    out_v[b, pc, pl.ds(pbg * LANES, LANES)] = pg
        for pbg, pc, pg in pend:
            out_v[b, pc, pl.ds(pbg * LANES, LANES)] = pg

    def pair_body(p, carry):
        for r in range(NB_OUT):
            n = p * NB_OUT + r
            pl.when(p > 0)(lambda r=r: wait_out(r))
            compute_n(n, r)
            fire_out(r, n)
        return carry

    lax.fori_loop(0, N_COLS // NB_OUT, pair_body, 0)
    for b in range(NB_OUT):
        wait_out(b)


def kernel(degree, degree_encoder_weight):
    idx_t = degree.T
    tab_t = degree_encoder_weight.T
    tab_pad = jnp.pad(tab_t, ((0, 0), (0, V_PAD - NUM_DEGREE)))
    tab_flat = (tab_pad.reshape(8, 8, 8, 128)
                .transpose(0, 2, 1, 3).reshape(TAB_WORDS))
    res = _gather_kernel(idx_t, tab_flat)
    return res.transpose(2, 0, 1)


# speedup vs baseline: 4.3043x; 1.0570x over previous
import functools

import jax
import jax.numpy as jnp
from jax import lax
from jax.experimental import pallas as pl
from jax.experimental.pallas import tpu as pltpu
from jax.experimental.pallas import tpu_sc as plsc

NUM_DEGREE = 1000
D_MODEL = 64
N_ROWS, N_COLS = 4096, 200
NC, NS = 2, 16
NW = NC * NS
BW = N_ROWS // NW
V_PAD = 1024
TAB_WORDS = D_MODEL * V_PAD
LANES = 16
N_BG = BW // LANES
NB_OUT = 2


@functools.partial(
    pl.kernel,
    out_type=jax.ShapeDtypeStruct((N_COLS, D_MODEL, N_ROWS), jnp.float32),
    mesh=plsc.VectorSubcoreMesh(core_axis_name="c", subcore_axis_name="s"),
    compiler_params=pltpu.CompilerParams(
        use_tc_tiling_on_sc=True, needs_layout_passes=False),
    scratch_types=[
        pltpu.VMEM((N_COLS, BW), jnp.int32),
        pltpu.VMEM((TAB_WORDS,), jnp.float32),
        pltpu.VMEM((NB_OUT, D_MODEL, BW), jnp.float32),
        pltpu.SemaphoreType.DMA,
        pltpu.SemaphoreType.DMA,
        [pltpu.SemaphoreType.DMA] * NB_OUT,
    ],
)
def _gather_kernel(idx_hbm, tab_hbm, out_hbm, idx_all, table_v, out_v,
                   sem_idx, sem_tab, sem_out):
    wid = lax.axis_index("s") * NC + lax.axis_index("c")
    bw = wid * BW

    cp_idx = pltpu.make_async_copy(
        idx_hbm.at[pl.ds(0, N_COLS), pl.ds(bw, BW)], idx_all, sem_idx)
    cp_tab = pltpu.make_async_copy(tab_hbm, table_v, sem_tab)
    cp_idx.start()
    cp_tab.start()
    cp_idx.wait()
    cp_tab.wait()

    def fire_out(b, n):
        pltpu.async_copy(
            out_v.at[b],
            out_hbm.at[n].at[pl.ds(0, D_MODEL), pl.ds(bw, BW)],
            sem_out[b])

    def wait_out(b):
        pltpu.make_async_copy(
            out_v.at[b],
            out_hbm.at[0].at[pl.ds(0, D_MODEL), pl.ds(bw, BW)],
            sem_out[b]).wait()

    def compute_n(n, b):
        bases = []
        for bg in range(N_BG):
            vi = idx_all[n, pl.ds(bg * LANES, LANES)]
            bases.append(((vi >> 7) << 10) + (vi & 127))
        pend = []
        for bg in range(N_BG):
            for c in range(D_MODEL):
                kc = (c // 8) * 8192 + (c % 8) * 128
                g = plsc.load_gather(
                    table_v.at[pl.ds(kc, TAB_WORDS - kc)], [bases[bg]])
                pend.append((bg, c, g))
                if len(pend) > LANES:
                    pbg, pc, pg = pend.pop(0)
                    out_v[b, pc, pl.ds(pbg * LANES, LANES)] = pg
        for pbg, pc, pg in pend:
            out_v[b, pc, pl.ds(pbg * LANES, LANES)] = pg

    def pair_body(p, carry):
        for r in range(NB_OUT):
            n = p * NB_OUT + r
            pl.when(p > 0)(lambda r=r: wait_out(r))
            compute_n(n, r)
            fire_out(r, n)
        return carry

    lax.fori_loop(0, N_COLS // NB_OUT, pair_body, 0)
    for b in range(NB_OUT):
        wait_out(b)


def kernel(degree, degree_encoder_weight):
    idx_t = degree.T
    tab_t = degree_encoder_weight.T
    tab_pad = jnp.pad(tab_t, ((0, 0), (0, V_PAD - NUM_DEGREE)))
    tab_flat = (tab_pad.reshape(8, 8, 8, 128)
                .transpose(0, 2, 1, 3).reshape(TAB_WORDS))
    res = _gather_kernel(idx_t, tab_flat)
    return res.transpose(2, 0, 1)


# speedup vs baseline: 6.2132x; 1.4435x over previous
import functools

import jax
import jax.numpy as jnp
from jax import lax
from jax.experimental import pallas as pl
from jax.experimental.pallas import tpu as pltpu
from jax.experimental.pallas import tpu_sc as plsc

NUM_DEGREE = 1000
D_MODEL = 64
N_ROWS, N_COLS = 4096, 200
NC, NS = 2, 16
NW = NC * NS
BW = N_ROWS // NW
V_PAD = 1024
TAB_WORDS = D_MODEL * V_PAD
LANES = 16
N_BG = BW // LANES
NB_OUT = 2


@functools.partial(
    pl.kernel,
    out_type=jax.ShapeDtypeStruct((N_COLS, D_MODEL, N_ROWS), jnp.float32),
    mesh=plsc.VectorSubcoreMesh(core_axis_name="c", subcore_axis_name="s"),
    compiler_params=pltpu.CompilerParams(
        use_tc_tiling_on_sc=True, needs_layout_passes=False),
    scratch_types=[
        pltpu.VMEM((N_COLS, BW), jnp.int32),
        pltpu.VMEM((TAB_WORDS,), jnp.float32),
        pltpu.VMEM((NB_OUT, D_MODEL, BW), jnp.float32),
        pltpu.SemaphoreType.DMA,
        pltpu.SemaphoreType.DMA,
        [pltpu.SemaphoreType.DMA] * NB_OUT,
    ],
)
def _gather_kernel(idx_hbm, tab_hbm, out_hbm, idx_all, table_v, out_v,
                   sem_idx, sem_tab, sem_out):
    wid = lax.axis_index("s") * NC + lax.axis_index("c")
    bw = wid * BW

    cp_idx = pltpu.make_async_copy(
        idx_hbm.at[pl.ds(0, N_COLS), pl.ds(bw, BW)], idx_all, sem_idx)
    cp_tab = pltpu.make_async_copy(tab_hbm, table_v, sem_tab)
    cp_idx.start()
    cp_tab.start()
    cp_idx.wait()
    cp_tab.wait()

    HALF = D_MODEL // 2

    def fire_out_half(b, n, h):
        pltpu.async_copy(
            out_v.at[b].at[pl.ds(h * HALF, HALF), pl.ds(0, BW)],
            out_hbm.at[n].at[pl.ds(h * HALF, HALF), pl.ds(bw, BW)],
            sem_out[b])

    def wait_out(b):
        for h in range(2):
            pltpu.make_async_copy(
                out_v.at[b].at[pl.ds(h * HALF, HALF), pl.ds(0, BW)],
                out_hbm.at[0].at[pl.ds(h * HALF, HALF), pl.ds(bw, BW)],
                sem_out[b]).wait()

    def compute_n(n, b):
        bases = []
        for bg in range(N_BG):
            vi = idx_all[n, pl.ds(bg * LANES, LANES)]
            bases.append(((vi >> 7) << 10) + (vi & 127))
        fire_at = HALF * N_BG - 1
        pend, stores = [], 0
        for c in range(D_MODEL):
            for bg in range(N_BG):
                kc = (c // 8) * 8192 + (c % 8) * 128
                g = plsc.load_gather(
                    table_v.at[pl.ds(kc, TAB_WORDS - kc)], [bases[bg]])
                pend.append((bg, c, g))
                if len(pend) > LANES:
                    pbg, pc, pg = pend.pop(0)
                    out_v[b, pc, pl.ds(pbg * LANES, LANES)] = pg
                    stores += 1
                    if stores == fire_at + 1:
                        fire_out_half(b, n, 0)
        for pbg, pc, pg in pend:
            out_v[b, pc, pl.ds(pbg * LANES, LANES)] = pg

    def pair_body(p, carry):
        for r in range(NB_OUT):
            n = p * NB_OUT + r
            pl.when(p > 0)(lambda r=r: wait_out(r))
            compute_n(n, r)
            fire_out_half(r, n, 1)
        return carry

    lax.fori_loop(0, N_COLS // NB_OUT, pair_body, 0)
    for b in range(NB_OUT):
        wait_out(b)


def kernel(degree, degree_encoder_weight):
    idx_t = degree.T
    tab_t = degree_encoder_weight.T
    tab_pad = jnp.pad(tab_t, ((0, 0), (0, V_PAD - NUM_DEGREE)))
    tab_flat = (tab_pad.reshape(8, 8, 8, 128)
                .transpose(0, 2, 1, 3).reshape(TAB_WORDS))
    res = _gather_kernel(idx_t, tab_flat)
    return res.transpose(2, 0, 1)
